# baseline (speedup 1.0000x reference)
import functools

import jax
import jax.numpy as jnp
from jax import lax
from jax.experimental import pallas as pl
from jax.experimental.pallas import tpu as pltpu
from jax.experimental.pallas import tpu_sc as plsc

NUM_CLASSES = 10
K = 5
MEMORY_SIZE = 50000
N_SAMP = 10000
N_PAD = 10240
QB = 256

_SENTINEL = 3.4028235e38


def _knn_block_kernel(xf_ref, xn_ref, memT_ref, aux_ref, eye_ref, out_ref):
    xq = xf_ref[...]
    memT = memT_ref[...]
    aux = aux_ref[...]
    xn = xn_ref[...]
    yn = aux[0:1, :]
    lab = aux[1:2, :]

    mm = jnp.dot(xq, memT)
    v = (xn + yn) - 2.0 * mm

    col = lax.broadcasted_iota(jnp.int32, (QB, N_PAD), 1)
    for _ in range(K):
        m = jnp.min(v, axis=1, keepdims=True)
        masked_iota = jnp.where(v == m, col, jnp.int32(N_PAD))
        idx = jnp.min(masked_iota, axis=1, keepdims=True)
        v = jnp.where(col == idx, _SENTINEL, v)

    sel = (v == _SENTINEL).astype(jnp.float32)

    cls16 = lax.broadcasted_iota(jnp.int32, (16, N_PAD), 0)
    onehotT = (lab.astype(jnp.int32) == cls16).astype(jnp.float32)
    counts = lax.dot_general(sel, onehotT,
                             (((1,), (1,)), ((), ())))

    maxc = jnp.max(counts, axis=1, keepdims=True)
    cls_row = lax.broadcasted_iota(jnp.int32, (QB, 16), 1)
    pred = jnp.min(jnp.where(counts == maxc, cls_row, jnp.int32(16)),
                   axis=1, keepdims=True)
    oh = (cls_row == pred).astype(jnp.float32)
    out16 = jnp.dot(oh, eye_ref[...])
    out_ref[...] = out16[:, :NUM_CLASSES]


def _run_tc(xf, xn, memT, aux, eye16):
    n = xf.shape[0]
    grid = n // QB
    return pl.pallas_call(
        _knn_block_kernel,
        grid=(grid,),
        in_specs=[
            pl.BlockSpec((QB, 16), lambda i: (i, 0)),
            pl.BlockSpec((QB, 1), lambda i: (i, 0)),
            pl.BlockSpec((16, N_PAD), lambda i: (0, 0)),
            pl.BlockSpec((8, N_PAD), lambda i: (0, 0)),
            pl.BlockSpec((16, 16), lambda i: (0, 0)),
        ],
        out_specs=pl.BlockSpec((QB, NUM_CLASSES), lambda i: (i, 0)),
        out_shape=jax.ShapeDtypeStruct((n, NUM_CLASSES), jnp.float32),
        compiler_params=pltpu.CompilerParams(
            dimension_semantics=("parallel",)),
    )(xf, xn, memT, aux, eye16)


_NW = 32
_BPW = N_PAD // _NW


def _sc_gather(memx, memy_flat, idx_pad):
    mesh = plsc.VectorSubcoreMesh(core_axis_name="c", subcore_axis_name="s")

    @functools.partial(
        pl.kernel, mesh=mesh,
        compiler_params=pltpu.CompilerParams(use_tc_tiling_on_sc=False),
        out_type=[jax.ShapeDtypeStruct((N_PAD, 16), jnp.float32),
                  jax.ShapeDtypeStruct((N_PAD,), jnp.int32)],
        scratch_types=[pltpu.VMEM((_BPW,), jnp.int32),
                       pltpu.VMEM((_BPW, 16), jnp.float32),
                       pltpu.VMEM((_BPW,), jnp.int32),
                       pltpu.SemaphoreType.DMA],
    )
    def gather_k(memx_hbm, memy_hbm, idx_hbm, outx_hbm, outy_hbm,
                 idx_v, rows_v, lab_v, sem):
        wid = lax.axis_index("s") * 2 + lax.axis_index("c")
        base = wid * _BPW
        pltpu.sync_copy(idx_hbm.at[pl.ds(base, _BPW)], idx_v)
        pltpu.async_copy(memx_hbm.at[idx_v], rows_v, sem).wait()
        pltpu.sync_copy(rows_v, outx_hbm.at[pl.ds(base, _BPW)])
        pltpu.async_copy(memy_hbm.at[idx_v], lab_v, sem).wait()
        pltpu.sync_copy(lab_v, outy_hbm.at[pl.ds(base, _BPW)])

    return gather_k(memx, memy_flat, idx_pad)


def kernel(x, y, memory_x, memory_y, eye):
    b, c, h, w = x.shape
    xf = jnp.transpose(x, (0, 2, 3, 1)).reshape(b * h * w, c)
    n = xf.shape[0]

    n_samp = min(MEMORY_SIZE, N_SAMP)
    mem_idx = jax.random.randint(jax.random.key(1234), (n_samp,), 0, n,
                                 dtype=jnp.int32)

    pad = N_PAD - n_samp
    idx_pad = jnp.concatenate([mem_idx, jnp.zeros((pad,), jnp.int32)])
    rows, lab_p = _sc_gather(memory_x, memory_y[:, 0], idx_pad)

    xn = jnp.sum(xf ** 2, axis=1).reshape(-1, 1)
    yn = jnp.sum(rows[:n_samp] ** 2, axis=1)

    memT = rows.T
    yn_p = jnp.concatenate([yn, jnp.full((pad,), jnp.inf, jnp.float32)])
    aux = jnp.zeros((8, N_PAD), jnp.float32)
    aux = aux.at[0, :].set(yn_p)
    aux = aux.at[1, :].set(lab_p.astype(jnp.float32))

    eye16 = jnp.zeros((16, 16), jnp.float32).at[:NUM_CLASSES, :NUM_CLASSES].set(eye)

    out2d = _run_tc(xf, xn, memT, aux, eye16)
    return jnp.transpose(out2d.reshape(b, h, w, NUM_CLASSES), (0, 3, 1, 2))


# speedup vs baseline: 1.8327x; 1.8327x over previous
import functools

import jax
import jax.numpy as jnp
from jax import lax
from jax.experimental import pallas as pl
from jax.experimental.pallas import tpu as pltpu
from jax.experimental.pallas import tpu_sc as plsc

NUM_CLASSES = 10
K = 5
MEMORY_SIZE = 50000
N_SAMP = 10000
N_PAD = 10240
QB = 256

_SENTINEL = 3.4028235e38


def _knn_block_kernel(xf_ref, xn_ref, memT_ref, aux_ref, eye_ref, out_ref):
    xq = xf_ref[...]
    memT = memT_ref[...]
    aux = aux_ref[...]
    xn = xn_ref[...]
    yn = aux[0:1, :]
    lab = aux[1:2, :]

    mm = jnp.dot(xq, memT)
    v = (xn + yn) - 2.0 * mm

    def _ce(a, bb):
        return jnp.minimum(a, bb), jnp.maximum(a, bb)

    s = [v[:, i * 128:(i + 1) * 128] for i in range(5)]
    for (i, j) in ((0, 1), (3, 4), (2, 4), (2, 3), (1, 4),
                   (0, 3), (0, 2), (1, 3), (1, 2)):
        s[i], s[j] = _ce(s[i], s[j])
    for p in range(5, 80):
        xpl = v[:, p * 128:(p + 1) * 128]
        for j in range(5):
            s[j], xpl = _ce(s[j], xpl)

    cand = jnp.concatenate(s, axis=1)
    col = lax.broadcasted_iota(jnp.int32, (QB, 640), 1)
    for _ in range(K - 1):
        m = jnp.min(cand, axis=1, keepdims=True)
        masked_iota = jnp.where(cand == m, col, jnp.int32(640))
        idx = jnp.min(masked_iota, axis=1, keepdims=True)
        cand = jnp.where(col == idx, _SENTINEL, cand)
    t5 = jnp.min(cand, axis=1, keepdims=True)

    below = (v < t5).astype(jnp.float32)
    eqm = (v == t5).astype(jnp.float32)

    cls16 = lax.broadcasted_iota(jnp.int32, (16, N_PAD), 0)
    onehotT = (lab.astype(jnp.int32) == cls16).astype(jnp.float32)
    dn = (((1,), (1,)), ((), ()))
    counts_b = lax.dot_general(below, onehotT, dn)
    counts_e = lax.dot_general(eqm, onehotT, dn)

    nb = jnp.sum(counts_b, axis=1, keepdims=True)
    cls_row16 = lax.broadcasted_iota(jnp.int32, (QB, 16), 1)
    lbound = jnp.min(jnp.where(counts_e > 0.0, cls_row16, jnp.int32(16)),
                     axis=1, keepdims=True)
    counts = counts_b + jnp.where(cls_row16 == lbound, 5.0 - nb, 0.0)

    maxc = jnp.max(counts, axis=1, keepdims=True)
    cls_row = lax.broadcasted_iota(jnp.int32, (QB, 16), 1)
    pred = jnp.min(jnp.where(counts == maxc, cls_row, jnp.int32(16)),
                   axis=1, keepdims=True)
    oh = (cls_row == pred).astype(jnp.float32)
    out16 = jnp.dot(oh, eye_ref[...])
    out_ref[...] = out16[:, :NUM_CLASSES]


def _run_tc(xf, xn, memT, aux, eye16):
    n = xf.shape[0]
    grid = n // QB
    return pl.pallas_call(
        _knn_block_kernel,
        grid=(grid,),
        in_specs=[
            pl.BlockSpec((QB, 16), lambda i: (i, 0)),
            pl.BlockSpec((QB, 1), lambda i: (i, 0)),
            pl.BlockSpec((16, N_PAD), lambda i: (0, 0)),
            pl.BlockSpec((8, N_PAD), lambda i: (0, 0)),
            pl.BlockSpec((16, 16), lambda i: (0, 0)),
        ],
        out_specs=pl.BlockSpec((QB, NUM_CLASSES), lambda i: (i, 0)),
        out_shape=jax.ShapeDtypeStruct((n, NUM_CLASSES), jnp.float32),
        compiler_params=pltpu.CompilerParams(
            dimension_semantics=("parallel",)),
    )(xf, xn, memT, aux, eye16)


_NW = 32
_BPW = N_PAD // _NW


def _sc_gather(memx, memy_flat, idx_pad):
    mesh = plsc.VectorSubcoreMesh(core_axis_name="c", subcore_axis_name="s")

    @functools.partial(
        pl.kernel, mesh=mesh,
        compiler_params=pltpu.CompilerParams(use_tc_tiling_on_sc=False),
        out_type=[jax.ShapeDtypeStruct((N_PAD, 16), jnp.float32),
                  jax.ShapeDtypeStruct((N_PAD,), jnp.int32)],
        scratch_types=[pltpu.VMEM((_BPW,), jnp.int32),
                       pltpu.VMEM((_BPW, 16), jnp.float32),
                       pltpu.VMEM((_BPW,), jnp.int32),
                       pltpu.SemaphoreType.DMA],
    )
    def gather_k(memx_hbm, memy_hbm, idx_hbm, outx_hbm, outy_hbm,
                 idx_v, rows_v, lab_v, sem):
        wid = lax.axis_index("s") * 2 + lax.axis_index("c")
        base = wid * _BPW
        pltpu.sync_copy(idx_hbm.at[pl.ds(base, _BPW)], idx_v)
        pltpu.async_copy(memx_hbm.at[idx_v], rows_v, sem).wait()
        pltpu.sync_copy(rows_v, outx_hbm.at[pl.ds(base, _BPW)])
        pltpu.async_copy(memy_hbm.at[idx_v], lab_v, sem).wait()
        pltpu.sync_copy(lab_v, outy_hbm.at[pl.ds(base, _BPW)])

    return gather_k(memx, memy_flat, idx_pad)


def kernel(x, y, memory_x, memory_y, eye):
    b, c, h, w = x.shape
    xf = jnp.transpose(x, (0, 2, 3, 1)).reshape(b * h * w, c)
    n = xf.shape[0]

    n_samp = min(MEMORY_SIZE, N_SAMP)
    mem_idx = jax.random.randint(jax.random.key(1234), (n_samp,), 0, n,
                                 dtype=jnp.int32)

    pad = N_PAD - n_samp
    idx_pad = jnp.concatenate([mem_idx, jnp.zeros((pad,), jnp.int32)])
    rows, lab_p = _sc_gather(memory_x, memory_y[:, 0], idx_pad)

    xn = jnp.sum(xf ** 2, axis=1).reshape(-1, 1)
    yn = jnp.sum(rows[:n_samp] ** 2, axis=1)

    memT = rows.T
    yn_p = jnp.concatenate([yn, jnp.full((pad,), jnp.inf, jnp.float32)])
    aux = jnp.zeros((8, N_PAD), jnp.float32)
    aux = aux.at[0, :].set(yn_p)
    aux = aux.at[1, :].set(lab_p.astype(jnp.float32))

    eye16 = jnp.zeros((16, 16), jnp.float32).at[:NUM_CLASSES, :NUM_CLASSES].set(eye)

    out2d = _run_tc(xf, xn, memT, aux, eye16)
    return jnp.transpose(out2d.reshape(b, h, w, NUM_CLASSES), (0, 3, 1, 2))


# speedup vs baseline: 2.0067x; 1.0949x over previous
import functools

import jax
import jax.numpy as jnp
from jax import lax
from jax.experimental import pallas as pl
from jax.experimental.pallas import tpu as pltpu
from jax.experimental.pallas import tpu_sc as plsc

NUM_CLASSES = 10
K = 5
MEMORY_SIZE = 50000
N_SAMP = 10000
N_PAD = 10240
QB = 256

_SENTINEL = 3.4028235e38


def _knn_block_kernel(xf_ref, xn_ref, memT_ref, aux_ref, eye_ref, out_ref):
    xq = xf_ref[...]
    memT = memT_ref[...]
    aux = aux_ref[...]
    xn = xn_ref[...]
    yn = aux[0:1, :]
    lab = aux[1:2, :]

    mm = jnp.dot(xq, memT)
    v = (xn + yn) - 2.0 * mm

    def _ce(a, bb):
        return jnp.minimum(a, bb), jnp.maximum(a, bb)

    def _merge5(a, b, resort=True):
        L = [jnp.minimum(a[i], b[4 - i]) for i in range(5)]
        if resort:
            for (i, j) in ((0, 4), (0, 2), (1, 3), (0, 1), (2, 3)):
                L[i], L[j] = _ce(L[i], L[j])
        return L

    _SORT8 = ((0, 1), (2, 3), (0, 2), (1, 3), (1, 2),
              (4, 5), (6, 7), (4, 6), (5, 7), (5, 6),
              (0, 4), (1, 5), (2, 6), (3, 7), (2, 4), (3, 5),
              (1, 2), (3, 4), (5, 6))

    groups = []
    for g in range(10):
        p = [v[:, (g * 8 + k) * 128:(g * 8 + k + 1) * 128] for k in range(8)]
        for (i, j) in _SORT8:
            p[i], p[j] = _ce(p[i], p[j])
        groups.append(p[:5])
    while len(groups) > 1:
        ng = [_merge5(a, b, resort=(len(groups) > 2))
              for a, b in zip(groups[0::2], groups[1::2])]
        if len(groups) % 2:
            ng.append(groups[-1])
        groups = ng
    s = groups[0]

    cand = jnp.concatenate(s, axis=1)
    col = lax.broadcasted_iota(jnp.int32, (QB, 640), 1)
    for _ in range(K - 1):
        m = jnp.min(cand, axis=1, keepdims=True)
        masked_iota = jnp.where(cand == m, col, jnp.int32(640))
        idx = jnp.min(masked_iota, axis=1, keepdims=True)
        cand = jnp.where(col == idx, _SENTINEL, cand)
    t5 = jnp.min(cand, axis=1, keepdims=True)

    below = (v < t5).astype(jnp.float32)
    eqm = (v == t5).astype(jnp.float32)

    cls16 = lax.broadcasted_iota(jnp.int32, (16, N_PAD), 0)
    onehotT = (lab.astype(jnp.int32) == cls16).astype(jnp.float32)
    dn = (((1,), (1,)), ((), ()))
    counts_b = lax.dot_general(below, onehotT, dn)
    counts_e = lax.dot_general(eqm, onehotT, dn)

    nb = jnp.sum(counts_b, axis=1, keepdims=True)
    cls_row16 = lax.broadcasted_iota(jnp.int32, (QB, 16), 1)
    lbound = jnp.min(jnp.where(counts_e > 0.0, cls_row16, jnp.int32(16)),
                     axis=1, keepdims=True)
    counts = counts_b + jnp.where(cls_row16 == lbound, 5.0 - nb, 0.0)

    maxc = jnp.max(counts, axis=1, keepdims=True)
    cls_row = lax.broadcasted_iota(jnp.int32, (QB, 16), 1)
    pred = jnp.min(jnp.where(counts == maxc, cls_row, jnp.int32(16)),
                   axis=1, keepdims=True)
    oh = (cls_row == pred).astype(jnp.float32)
    out16 = jnp.dot(oh, eye_ref[...])
    out_ref[...] = out16[:, :NUM_CLASSES]


def _run_tc(xf, xn, memT, aux, eye16):
    n = xf.shape[0]
    grid = n // QB
    return pl.pallas_call(
        _knn_block_kernel,
        grid=(grid,),
        in_specs=[
            pl.BlockSpec((QB, 16), lambda i: (i, 0)),
            pl.BlockSpec((QB, 1), lambda i: (i, 0)),
            pl.BlockSpec((16, N_PAD), lambda i: (0, 0)),
            pl.BlockSpec((8, N_PAD), lambda i: (0, 0)),
            pl.BlockSpec((16, 16), lambda i: (0, 0)),
        ],
        out_specs=pl.BlockSpec((QB, NUM_CLASSES), lambda i: (i, 0)),
        out_shape=jax.ShapeDtypeStruct((n, NUM_CLASSES), jnp.float32),
        compiler_params=pltpu.CompilerParams(
            dimension_semantics=("parallel",)),
    )(xf, xn, memT, aux, eye16)


_NW = 32
_BPW = N_PAD // _NW


def _sc_gather(memx, memy_flat, idx_pad):
    mesh = plsc.VectorSubcoreMesh(core_axis_name="c", subcore_axis_name="s")

    @functools.partial(
        pl.kernel, mesh=mesh,
        compiler_params=pltpu.CompilerParams(use_tc_tiling_on_sc=False),
        out_type=[jax.ShapeDtypeStruct((N_PAD, 16), jnp.float32),
                  jax.ShapeDtypeStruct((N_PAD,), jnp.int32)],
        scratch_types=[pltpu.VMEM((_BPW,), jnp.int32),
                       pltpu.VMEM((_BPW, 16), jnp.float32),
                       pltpu.VMEM((_BPW,), jnp.int32),
                       pltpu.SemaphoreType.DMA],
    )
    def gather_k(memx_hbm, memy_hbm, idx_hbm, outx_hbm, outy_hbm,
                 idx_v, rows_v, lab_v, sem):
        wid = lax.axis_index("s") * 2 + lax.axis_index("c")
        base = wid * _BPW
        pltpu.sync_copy(idx_hbm.at[pl.ds(base, _BPW)], idx_v)
        pltpu.async_copy(memx_hbm.at[idx_v], rows_v, sem).wait()
        pltpu.sync_copy(rows_v, outx_hbm.at[pl.ds(base, _BPW)])
        pltpu.async_copy(memy_hbm.at[idx_v], lab_v, sem).wait()
        pltpu.sync_copy(lab_v, outy_hbm.at[pl.ds(base, _BPW)])

    return gather_k(memx, memy_flat, idx_pad)


def kernel(x, y, memory_x, memory_y, eye):
    b, c, h, w = x.shape
    xf = jnp.transpose(x, (0, 2, 3, 1)).reshape(b * h * w, c)
    n = xf.shape[0]

    n_samp = min(MEMORY_SIZE, N_SAMP)
    mem_idx = jax.random.randint(jax.random.key(1234), (n_samp,), 0, n,
                                 dtype=jnp.int32)

    pad = N_PAD - n_samp
    idx_pad = jnp.concatenate([mem_idx, jnp.zeros((pad,), jnp.int32)])
    rows, lab_p = _sc_gather(memory_x, memory_y[:, 0], idx_pad)

    xn = jnp.sum(xf ** 2, axis=1).reshape(-1, 1)
    yn = jnp.sum(rows[:n_samp] ** 2, axis=1)

    memT = rows.T
    yn_p = jnp.concatenate([yn, jnp.full((pad,), jnp.inf, jnp.float32)])
    aux = jnp.zeros((8, N_PAD), jnp.float32)
    aux = aux.at[0, :].set(yn_p)
    aux = aux.at[1, :].set(lab_p.astype(jnp.float32))

    eye16 = jnp.zeros((16, 16), jnp.float32).at[:NUM_CLASSES, :NUM_CLASSES].set(eye)

    out2d = _run_tc(xf, xn, memT, aux, eye16)
    return jnp.transpose(out2d.reshape(b, h, w, NUM_CLASSES), (0, 3, 1, 2))


# speedup vs baseline: 2.0494x; 1.0212x over previous
import functools

import jax
import jax.numpy as jnp
from jax import lax
from jax.experimental import pallas as pl
from jax.experimental.pallas import tpu as pltpu
from jax.experimental.pallas import tpu_sc as plsc

NUM_CLASSES = 10
K = 5
MEMORY_SIZE = 50000
N_SAMP = 10000
N_PAD = 10240
QB = 256

_SENTINEL = 3.4028235e38


def _knn_block_kernel(xf_ref, xn_ref, memT_ref, aux_ref, eye_ref, out_ref):
    xq = xf_ref[...]
    memT = memT_ref[...]
    aux = aux_ref[...]
    xn = xn_ref[...]
    yn = aux[0:1, :]
    lab = aux[1:2, :]

    mm = jnp.dot(xq, memT)
    v = (xn + yn) + mm

    def _ce(a, bb):
        return jnp.minimum(a, bb), jnp.maximum(a, bb)

    def _merge5(a, b, resort=True):
        L = [jnp.minimum(a[i], b[4 - i]) for i in range(5)]
        if resort:
            for (i, j) in ((0, 4), (0, 2), (1, 3), (0, 1), (2, 3)):
                L[i], L[j] = _ce(L[i], L[j])
        return L

    _S8 = (("ce", 0, 1), ("ce", 2, 3), ("ce", 4, 5), ("ce", 6, 7),
           ("ce", 0, 2), ("ce", 1, 3), ("ce", 4, 6), ("ce", 5, 7),
           ("ce", 1, 2), ("ce", 5, 6),
           ("ce", 0, 4), ("ce", 1, 5), ("min", 2, 6), ("min", 3, 7),
           ("ce", 2, 4), ("min", 3, 5),
           ("ce", 1, 2), ("ce", 3, 4))

    groups = []
    for g in range(10):
        p = [v[:, (g * 8 + k) * 128:(g * 8 + k + 1) * 128] for k in range(8)]
        for (kind, i, j) in _S8:
            if kind == "ce":
                p[i], p[j] = _ce(p[i], p[j])
            else:
                p[i] = jnp.minimum(p[i], p[j])
        groups.append(p[:5])
    while len(groups) > 1:
        ng = [_merge5(a, b, resort=(len(groups) > 2))
              for a, b in zip(groups[0::2], groups[1::2])]
        if len(groups) % 2:
            ng.append(groups[-1])
        groups = ng
    s = groups[0]

    cand = jnp.concatenate(s, axis=1)
    col = lax.broadcasted_iota(jnp.int32, (QB, 640), 1)
    for _ in range(K - 1):
        m = jnp.min(cand, axis=1, keepdims=True)
        masked_iota = jnp.where(cand == m, col, jnp.int32(640))
        idx = jnp.min(masked_iota, axis=1, keepdims=True)
        cand = jnp.where(col == idx, _SENTINEL, cand)
    t5 = jnp.min(cand, axis=1, keepdims=True)

    below = (v < t5).astype(jnp.float32)
    eqm = (v == t5).astype(jnp.float32)

    cls16 = lax.broadcasted_iota(jnp.int32, (16, N_PAD), 0)
    onehotT = (lab.astype(jnp.int32) == cls16).astype(jnp.float32)
    dn = (((1,), (1,)), ((), ()))
    counts_b = lax.dot_general(below, onehotT, dn)
    counts_e = lax.dot_general(eqm, onehotT, dn)

    nb = jnp.sum(counts_b, axis=1, keepdims=True)
    cls_row16 = lax.broadcasted_iota(jnp.int32, (QB, 16), 1)
    lbound = jnp.min(jnp.where(counts_e > 0.0, cls_row16, jnp.int32(16)),
                     axis=1, keepdims=True)
    counts = counts_b + jnp.where(cls_row16 == lbound, 5.0 - nb, 0.0)

    maxc = jnp.max(counts, axis=1, keepdims=True)
    cls_row = lax.broadcasted_iota(jnp.int32, (QB, 16), 1)
    pred = jnp.min(jnp.where(counts == maxc, cls_row, jnp.int32(16)),
                   axis=1, keepdims=True)
    oh = (cls_row == pred).astype(jnp.float32)
    out16 = jnp.dot(oh, eye_ref[...])
    out_ref[...] = out16[:, :NUM_CLASSES]


def _run_tc(xf, xn, memT, aux, eye16):
    n = xf.shape[0]
    grid = n // QB
    return pl.pallas_call(
        _knn_block_kernel,
        grid=(grid,),
        in_specs=[
            pl.BlockSpec((QB, 16), lambda i: (i, 0)),
            pl.BlockSpec((QB, 1), lambda i: (i, 0)),
            pl.BlockSpec((16, N_PAD), lambda i: (0, 0)),
            pl.BlockSpec((8, N_PAD), lambda i: (0, 0)),
            pl.BlockSpec((16, 16), lambda i: (0, 0)),
        ],
        out_specs=pl.BlockSpec((QB, NUM_CLASSES), lambda i: (i, 0)),
        out_shape=jax.ShapeDtypeStruct((n, NUM_CLASSES), jnp.float32),
        compiler_params=pltpu.CompilerParams(
            dimension_semantics=("parallel",)),
    )(xf, xn, memT, aux, eye16)


_NW = 32
_BPW = N_PAD // _NW


def _sc_gather(memx, memy_flat, idx_pad):
    mesh = plsc.VectorSubcoreMesh(core_axis_name="c", subcore_axis_name="s")

    @functools.partial(
        pl.kernel, mesh=mesh,
        compiler_params=pltpu.CompilerParams(use_tc_tiling_on_sc=False),
        out_type=[jax.ShapeDtypeStruct((N_PAD, 16), jnp.float32),
                  jax.ShapeDtypeStruct((N_PAD,), jnp.int32)],
        scratch_types=[pltpu.VMEM((_BPW,), jnp.int32),
                       pltpu.VMEM((_BPW, 16), jnp.float32),
                       pltpu.VMEM((_BPW,), jnp.int32),
                       pltpu.SemaphoreType.DMA],
    )
    def gather_k(memx_hbm, memy_hbm, idx_hbm, outx_hbm, outy_hbm,
                 idx_v, rows_v, lab_v, sem):
        wid = lax.axis_index("s") * 2 + lax.axis_index("c")
        base = wid * _BPW
        pltpu.sync_copy(idx_hbm.at[pl.ds(base, _BPW)], idx_v)
        pltpu.async_copy(memx_hbm.at[idx_v], rows_v, sem).wait()
        pltpu.sync_copy(rows_v, outx_hbm.at[pl.ds(base, _BPW)])
        pltpu.async_copy(memy_hbm.at[idx_v], lab_v, sem).wait()
        pltpu.sync_copy(lab_v, outy_hbm.at[pl.ds(base, _BPW)])

    return gather_k(memx, memy_flat, idx_pad)


def kernel(x, y, memory_x, memory_y, eye):
    b, c, h, w = x.shape
    xf = jnp.transpose(x, (0, 2, 3, 1)).reshape(b * h * w, c)
    n = xf.shape[0]

    n_samp = min(MEMORY_SIZE, N_SAMP)
    mem_idx = jax.random.randint(jax.random.key(1234), (n_samp,), 0, n,
                                 dtype=jnp.int32)

    pad = N_PAD - n_samp
    idx_pad = jnp.concatenate([mem_idx, jnp.zeros((pad,), jnp.int32)])
    rows, lab_p = _sc_gather(memory_x, memory_y[:, 0], idx_pad)

    xn = jnp.sum(xf ** 2, axis=1).reshape(-1, 1)
    yn = jnp.sum(rows[:n_samp] ** 2, axis=1)

    memT = -2.0 * rows.T
    yn_p = jnp.concatenate([yn, jnp.full((pad,), jnp.inf, jnp.float32)])
    aux = jnp.zeros((8, N_PAD), jnp.float32)
    aux = aux.at[0, :].set(yn_p)
    aux = aux.at[1, :].set(lab_p.astype(jnp.float32))

    eye16 = jnp.zeros((16, 16), jnp.float32).at[:NUM_CLASSES, :NUM_CLASSES].set(eye)

    out2d = _run_tc(xf, xn, memT, aux, eye16)
    return jnp.transpose(out2d.reshape(b, h, w, NUM_CLASSES), (0, 3, 1, 2))


# speedup vs baseline: 2.1913x; 1.0693x over previous
import functools

import jax
import jax.numpy as jnp
from jax import lax
from jax.experimental import pallas as pl
from jax.experimental.pallas import tpu as pltpu
from jax.experimental.pallas import tpu_sc as plsc

NUM_CLASSES = 10
K = 5
MEMORY_SIZE = 50000
N_SAMP = 10000
N_PAD = 10240
QB = 512

_SENTINEL = 3.4028235e38


def _knn_block_kernel(xf_ref, xn_ref, memT_ref, aux_ref, eye_ref, out_ref):
    xq = xf_ref[...]
    memT = memT_ref[...]
    aux = aux_ref[...]
    xn = xn_ref[...]
    yn = aux[0:1, :]
    lab = aux[1:2, :]

    mm = jnp.dot(xq, memT)
    v = (xn + yn) + mm

    def _ce(a, bb):
        return jnp.minimum(a, bb), jnp.maximum(a, bb)

    def _merge5(a, b, resort=True):
        L = [jnp.minimum(a[i], b[4 - i]) for i in range(5)]
        if resort:
            for (i, j) in ((0, 4), (0, 2), (1, 3), (0, 1), (2, 3)):
                L[i], L[j] = _ce(L[i], L[j])
        return L

    _S8 = (("ce", 0, 1), ("ce", 2, 3), ("ce", 4, 5), ("ce", 6, 7),
           ("ce", 0, 2), ("ce", 1, 3), ("ce", 4, 6), ("ce", 5, 7),
           ("ce", 1, 2), ("ce", 5, 6),
           ("ce", 0, 4), ("ce", 1, 5), ("min", 2, 6), ("min", 3, 7),
           ("ce", 2, 4), ("min", 3, 5),
           ("ce", 1, 2), ("ce", 3, 4))

    groups = []
    for g in range(10):
        p = [v[:, (g * 8 + k) * 128:(g * 8 + k + 1) * 128] for k in range(8)]
        for (kind, i, j) in _S8:
            if kind == "ce":
                p[i], p[j] = _ce(p[i], p[j])
            else:
                p[i] = jnp.minimum(p[i], p[j])
        groups.append(p[:5])
    while len(groups) > 1:
        ng = [_merge5(a, b, resort=(len(groups) > 2))
              for a, b in zip(groups[0::2], groups[1::2])]
        if len(groups) % 2:
            ng.append(groups[-1])
        groups = ng
    s = groups[0]

    cand = jnp.concatenate(s, axis=1)
    col = lax.broadcasted_iota(jnp.int32, (QB, 640), 1)
    for _ in range(K - 1):
        m = jnp.min(cand, axis=1, keepdims=True)
        masked_iota = jnp.where(cand == m, col, jnp.int32(640))
        idx = jnp.min(masked_iota, axis=1, keepdims=True)
        cand = jnp.where(col == idx, _SENTINEL, cand)
    t5 = jnp.min(cand, axis=1, keepdims=True)

    below = (v < t5).astype(jnp.float32)
    eqm = (v == t5).astype(jnp.float32)

    cls16 = lax.broadcasted_iota(jnp.int32, (16, N_PAD), 0)
    onehotT = (lab.astype(jnp.int32) == cls16).astype(jnp.float32)
    dn = (((1,), (1,)), ((), ()))
    counts_b = lax.dot_general(below, onehotT, dn)
    counts_e = lax.dot_general(eqm, onehotT, dn)

    nb = jnp.sum(counts_b, axis=1, keepdims=True)
    cls_row16 = lax.broadcasted_iota(jnp.int32, (QB, 16), 1)
    lbound = jnp.min(jnp.where(counts_e > 0.0, cls_row16, jnp.int32(16)),
                     axis=1, keepdims=True)
    counts = counts_b + jnp.where(cls_row16 == lbound, 5.0 - nb, 0.0)

    maxc = jnp.max(counts, axis=1, keepdims=True)
    cls_row = lax.broadcasted_iota(jnp.int32, (QB, 16), 1)
    pred = jnp.min(jnp.where(counts == maxc, cls_row, jnp.int32(16)),
                   axis=1, keepdims=True)
    oh = (cls_row == pred).astype(jnp.float32)
    out16 = jnp.dot(oh, eye_ref[...])
    out_ref[...] = out16[:, :NUM_CLASSES]


def _run_tc(xf, xn, memT, aux, eye16):
    n = xf.shape[0]
    grid = n // QB
    return pl.pallas_call(
        _knn_block_kernel,
        grid=(grid,),
        in_specs=[
            pl.BlockSpec((QB, 16), lambda i: (i, 0)),
            pl.BlockSpec((QB, 1), lambda i: (i, 0)),
            pl.BlockSpec((16, N_PAD), lambda i: (0, 0)),
            pl.BlockSpec((8, N_PAD), lambda i: (0, 0)),
            pl.BlockSpec((16, 16), lambda i: (0, 0)),
        ],
        out_specs=pl.BlockSpec((QB, NUM_CLASSES), lambda i: (i, 0)),
        out_shape=jax.ShapeDtypeStruct((n, NUM_CLASSES), jnp.float32),
        compiler_params=pltpu.CompilerParams(
            dimension_semantics=("parallel",)),
    )(xf, xn, memT, aux, eye16)


_NW = 32
_BPW = N_PAD // _NW


def _sc_gather(memx, memy_flat, idx_pad):
    mesh = plsc.VectorSubcoreMesh(core_axis_name="c", subcore_axis_name="s")

    @functools.partial(
        pl.kernel, mesh=mesh,
        compiler_params=pltpu.CompilerParams(use_tc_tiling_on_sc=False),
        out_type=[jax.ShapeDtypeStruct((N_PAD, 16), jnp.float32),
                  jax.ShapeDtypeStruct((N_PAD,), jnp.int32)],
        scratch_types=[pltpu.VMEM((_BPW,), jnp.int32),
                       pltpu.VMEM((_BPW, 16), jnp.float32),
                       pltpu.VMEM((_BPW,), jnp.int32),
                       pltpu.SemaphoreType.DMA],
    )
    def gather_k(memx_hbm, memy_hbm, idx_hbm, outx_hbm, outy_hbm,
                 idx_v, rows_v, lab_v, sem):
        wid = lax.axis_index("s") * 2 + lax.axis_index("c")
        base = wid * _BPW
        pltpu.sync_copy(idx_hbm.at[pl.ds(base, _BPW)], idx_v)
        pltpu.async_copy(memx_hbm.at[idx_v], rows_v, sem).wait()
        pltpu.sync_copy(rows_v, outx_hbm.at[pl.ds(base, _BPW)])
        pltpu.async_copy(memy_hbm.at[idx_v], lab_v, sem).wait()
        pltpu.sync_copy(lab_v, outy_hbm.at[pl.ds(base, _BPW)])

    return gather_k(memx, memy_flat, idx_pad)


def kernel(x, y, memory_x, memory_y, eye):
    b, c, h, w = x.shape
    xf = jnp.transpose(x, (0, 2, 3, 1)).reshape(b * h * w, c)
    n = xf.shape[0]

    n_samp = min(MEMORY_SIZE, N_SAMP)
    mem_idx = jax.random.randint(jax.random.key(1234), (n_samp,), 0, n,
                                 dtype=jnp.int32)

    pad = N_PAD - n_samp
    idx_pad = jnp.concatenate([mem_idx, jnp.zeros((pad,), jnp.int32)])
    rows, lab_p = _sc_gather(memory_x, memory_y[:, 0], idx_pad)

    xn = jnp.sum(xf ** 2, axis=1).reshape(-1, 1)
    yn = jnp.sum(rows[:n_samp] ** 2, axis=1)

    memT = -2.0 * rows.T
    yn_p = jnp.concatenate([yn, jnp.full((pad,), jnp.inf, jnp.float32)])
    aux = jnp.zeros((8, N_PAD), jnp.float32)
    aux = aux.at[0, :].set(yn_p)
    aux = aux.at[1, :].set(lab_p.astype(jnp.float32))

    eye16 = jnp.zeros((16, 16), jnp.float32).at[:NUM_CLASSES, :NUM_CLASSES].set(eye)

    out2d = _run_tc(xf, xn, memT, aux, eye16)
    return jnp.transpose(out2d.reshape(b, h, w, NUM_CLASSES), (0, 3, 1, 2))
